# baseline (speedup 1.0000x reference)
import functools

import jax
import jax.numpy as jnp
from jax import lax
from jax.experimental import pallas as pl
from jax.experimental.pallas import tpu as pltpu
from jax.experimental.pallas import tpu_sc as plsc

SEQ = 200
DIM = 32
NC = 2
NS = 16
NW = NC * NS

SPC = 4
CHUNK = SPC * SEQ
LANES = 16
NBLK = 13


def _emb_kernel(n_seq, idx_hbm, tok_hbm, pos_hbm, out_hbm,
                pos_v, idx_v, qidx_v, g0, g1, o0, o1, gs0, gs1, ws0, ws1):
    seq_per_w = n_seq // NW
    n_chunks = seq_per_w // SPC

    wid = lax.axis_index("s") * NC + lax.axis_index("c")
    seq0 = wid * seq_per_w
    sets = ((g0, o0, gs0, ws0), (g1, o1, gs1, ws1))

    pltpu.sync_copy(pos_hbm, pos_v)

    def fire_gather(sl, gbuf, gsem):
        pltpu.async_copy(tok_hbm.at[qidx_v.at[pl.ds(sl * SEQ, SEQ)]],
                         gbuf, gsem)

    def wait_gather(sl, gbuf, gsem):
        pltpu.make_async_copy(tok_hbm.at[qidx_v.at[pl.ds(sl * SEQ, SEQ)]],
                              gbuf, gsem).wait()

    def fire_wb(s_abs, obuf, wsem):
        pltpu.async_copy(obuf, out_hbm.at[s_abs], wsem)

    def wait_wb(s_abs, obuf, wsem):
        pltpu.make_async_copy(obuf, out_hbm.at[s_abs], wsem).wait()

    def select_seq(sl, gbuf, obuf):
        soff = sl * SEQ

        def blk_body(blk, carry):
            p0 = lax.min(blk * LANES, SEQ - LANES)
            idx_vec = idx_v[pl.ds(soff + p0, LANES)]
            for l in range(LANES):
                off = (idx_vec[l] & 3) << 5
                p = p0 + l
                v0 = gbuf[p, pl.ds(off, 16)] + pos_v[p, pl.ds(0, 16)]
                v1 = gbuf[p, pl.ds(off + 16, 16)] + pos_v[p, pl.ds(16, 16)]
                obuf[p, pl.ds(0, 16)] = v0
                obuf[p, pl.ds(16, 16)] = v1
            return carry

        lax.fori_loop(0, NBLK, blk_body, 0)

    def chunk_body(c, carry):
        sbase = seq0 + c * SPC
        pltpu.sync_copy(idx_hbm.at[pl.ds(sbase * SEQ, CHUNK)], idx_v)

        @plsc.parallel_loop(0, CHUNK // LANES, unroll=8)
        def _q(j):
            qidx_v[pl.ds(j * LANES, LANES)] = lax.shift_right_logical(
                idx_v[pl.ds(j * LANES, LANES)], 2)

        fire_gather(0, g0, gs0)

        def pair_body(i, c2):
            for b in (0, 1):
                gbuf, obuf, gsem, wsem = sets[b]
                ngbuf, _, ngsem, _ = sets[1 - b]
                sl = 2 * i + b

                @pl.when(sl + 1 < SPC)
                def _fire():
                    fire_gather(sl + 1, ngbuf, ngsem)

                wait_gather(sl, gbuf, gsem)

                @pl.when(jnp.logical_or(sl >= 2, c >= 1))
                def _wb_done():
                    wait_wb(sbase + sl - 2, obuf, wsem)

                select_seq(sl, gbuf, obuf)
                fire_wb(sbase + sl, obuf, wsem)
            return c2

        lax.fori_loop(0, SPC // 2, pair_body, 0)
        return carry

    lax.fori_loop(0, n_chunks, chunk_body, 0)
    last = seq0 + n_chunks * SPC
    wait_wb(last - 2, o0, ws0)
    wait_wb(last - 1, o1, ws1)


def kernel(inputs, token_table, pos_table):
    b, s = inputs.shape
    n_rows = b * s
    assert s == SEQ and token_table.shape[1] == DIM
    assert b % (NW * SPC) == 0 and token_table.shape[0] % 4 == 0

    idx = inputs.reshape(n_rows).astype(jnp.int32)
    tok_4 = token_table.reshape(token_table.size // 128, 128)

    mesh = plsc.VectorSubcoreMesh(core_axis_name="c", subcore_axis_name="s")
    k = functools.partial(
        pl.kernel,
        mesh=mesh,
        out_type=jax.ShapeDtypeStruct((b, s, DIM), jnp.float32),
        scratch_types=[
            pltpu.VMEM((SEQ, DIM), jnp.float32),
            pltpu.VMEM((CHUNK,), jnp.int32),
            pltpu.VMEM((CHUNK,), jnp.int32),
            pltpu.VMEM((SEQ, 128), jnp.float32),
            pltpu.VMEM((SEQ, 128), jnp.float32),
            pltpu.VMEM((SEQ, DIM), jnp.float32),
            pltpu.VMEM((SEQ, DIM), jnp.float32),
            pltpu.SemaphoreType.DMA,
            pltpu.SemaphoreType.DMA,
            pltpu.SemaphoreType.DMA,
            pltpu.SemaphoreType.DMA,
        ],
    )(functools.partial(_emb_kernel, b))

    return k(idx, tok_4, pos_table)


# speedup vs baseline: 1.0747x; 1.0747x over previous
import functools

import jax
import jax.numpy as jnp
from jax import lax
from jax.experimental import pallas as pl
from jax.experimental.pallas import tpu as pltpu
from jax.experimental.pallas import tpu_sc as plsc

SEQ = 200
DIM = 32
NC = 2
NS = 16
NW = NC * NS

CS = 8
R = CS * SEQ
G = 64
NG = R // G


def _emb_kernel(n_rows, idx_hbm, tok_hbm, pos_hbm, out_hbm,
                pos_v, idx0, idx1, buf0, buf1, gs0, gs1, ws0, ws1):
    rows_per_w = n_rows // NW
    n_chunks = rows_per_w // R

    wid = lax.axis_index("s") * NC + lax.axis_index("c")
    base0 = wid * rows_per_w
    sets = ((idx0, buf0, gs0, ws0), (idx1, buf1, gs1, ws1))

    def fire(c, idxr, bufr, gsem):
        base = base0 + c * R
        pltpu.sync_copy(idx_hbm.at[pl.ds(base, R)], idxr)
        for g in range(NG):
            pltpu.async_copy(tok_hbm.at[idxr.at[pl.ds(g * G, G)]],
                             bufr.at[pl.ds(g * G, G)], gsem)

    def drain_gathers(idxr, bufr, gsem):
        for g in range(NG):
            pltpu.make_async_copy(tok_hbm.at[idxr.at[pl.ds(g * G, G)]],
                                  bufr.at[pl.ds(g * G, G)], gsem).wait()

    def wait_writeback(c, bufr, wsem):
        pltpu.make_async_copy(bufr, out_hbm.at[pl.ds(base0 + c * R, R)],
                              wsem).wait()

    pltpu.sync_copy(pos_hbm, pos_v)
    fire(0, idx0, buf0, gs0)

    def pair_body(i, carry):
        for b in (0, 1):
            idxr, bufr, gsem, wsem = sets[b]
            nidxr, nbufr, ngsem, nwsem = sets[1 - b]
            c = 2 * i + b

            @pl.when(c + 1 < n_chunks)
            def _fire_next():
                @pl.when(c >= 1)
                def _wb():
                    wait_writeback(c - 1, nbufr, nwsem)
                fire(c + 1, nidxr, nbufr, ngsem)

            drain_gathers(idxr, bufr, gsem)

            def seq_body(s, c2):
                @plsc.parallel_loop(0, SEQ, unroll=8)
                def _row(p):
                    row = s * SEQ + p
                    plsc.addupdate(bufr.at[row, pl.ds(0, 16)],
                                   pos_v[p, pl.ds(0, 16)])
                    plsc.addupdate(bufr.at[row, pl.ds(16, 16)],
                                   pos_v[p, pl.ds(16, 16)])
                return c2
            lax.fori_loop(0, CS, seq_body, 0)

            pltpu.async_copy(bufr, out_hbm.at[pl.ds(base0 + c * R, R)], wsem)
        return carry

    lax.fori_loop(0, n_chunks // 2, pair_body, 0)
    wait_writeback(n_chunks - 2, buf0, ws0)
    wait_writeback(n_chunks - 1, buf1, ws1)


def kernel(inputs, token_table, pos_table):
    b, s = inputs.shape
    n_rows = b * s
    assert s == SEQ and token_table.shape[1] == DIM
    assert n_rows % (NW * 2 * R) == 0

    idx = inputs.reshape(n_rows).astype(jnp.int32)

    mesh = plsc.VectorSubcoreMesh(core_axis_name="c", subcore_axis_name="s")
    k = functools.partial(
        pl.kernel,
        mesh=mesh,
        compiler_params=pltpu.CompilerParams(use_tc_tiling_on_sc=False),
        out_type=jax.ShapeDtypeStruct((n_rows, DIM), jnp.float32),
        scratch_types=[
            pltpu.VMEM((SEQ, DIM), jnp.float32),
            pltpu.VMEM((R,), jnp.int32),
            pltpu.VMEM((R,), jnp.int32),
            pltpu.VMEM((R, DIM), jnp.float32),
            pltpu.VMEM((R, DIM), jnp.float32),
            pltpu.SemaphoreType.DMA,
            pltpu.SemaphoreType.DMA,
            pltpu.SemaphoreType.DMA,
            pltpu.SemaphoreType.DMA,
        ],
    )(functools.partial(_emb_kernel, n_rows))

    out = k(idx, token_table, pos_table)
    return out.reshape(b, s, DIM)
